# baseline (speedup 1.0000x reference)
import functools

import jax
import jax.numpy as jnp
from jax import lax
from jax.experimental import pallas as pl
from jax.experimental.pallas import tpu as pltpu
from jax.experimental.pallas import tpu_sc as plsc

_N = 10000
_E = 320000
_D = 128
_H = 128
_C = 40

_NC = 2
_NS = 16
_NW = _NC * _NS
_K = 100

_RT_ZERO = 640
_NPAD = _NS * _RT_ZERO


_D = 2


def _make_edge_pass(w, k, with_counts):
    e_per_tile = _E // _NW
    n_chunks = e_per_tile // k
    assert n_chunks % _D == 0 and e_per_tile % k == 0
    mesh = plsc.VectorSubcoreMesh(core_axis_name="c", subcore_axis_name="s")

    out_type = [jax.ShapeDtypeStruct((_NC, _NPAD, w), jnp.float32)]
    scratch = [
        pltpu.VMEM_SHARED((_NPAD, w), jnp.float32),
        pltpu.VMEM((n_chunks, k), jnp.int32),
        pltpu.VMEM((n_chunks, k), jnp.int32),
    ]
    scratch += [pltpu.VMEM((k, w), jnp.float32) for _ in range(_D)]
    scratch += [pltpu.SemaphoreType.DMA for _ in range(2 * _D)]
    if with_counts:
        out_type.append(jax.ShapeDtypeStruct((_NC, _NPAD, 16), jnp.float32))
        scratch += [
            pltpu.VMEM_SHARED((_NPAD, 16), jnp.float32),
            pltpu.VMEM((k, 16), jnp.float32),
            pltpu.SemaphoreType.DMA,
            pltpu.SemaphoreType.DMA,
        ]

    @functools.partial(
        pl.kernel,
        out_type=out_type,
        mesh=mesh,
        scratch_types=scratch,
        compiler_params=pltpu.CompilerParams(use_tc_tiling_on_sc=False),
    )
    def edge_pass(table, src2, dst2, zrows, *rest):
        if with_counts:
            (zrows16, ones16, out, cout, acc, src_v, dst_v, *bufsems) = rest
            bufs = bufsems[:_D]
            gsem = bufsems[_D:2 * _D]
            ssem = bufsems[2 * _D:3 * _D]
            cacc, ones_v, csem0, csem1 = bufsems[3 * _D:]
            csem = [csem0, csem1]
        else:
            (out, acc, src_v, dst_v, *bufsems) = rest
            bufs = bufsems[:_D]
            gsem = bufsems[_D:2 * _D]
            ssem = bufsems[2 * _D:3 * _D]
        cid = lax.axis_index("c")
        sid = lax.axis_index("s")
        wid = cid * _NS + sid
        pltpu.sync_copy(zrows, acc.at[pl.ds(sid * _RT_ZERO, _RT_ZERO)])
        pltpu.sync_copy(src2.at[wid], src_v)
        pltpu.sync_copy(dst2.at[wid], dst_v)
        if with_counts:
            pltpu.sync_copy(zrows16, cacc.at[pl.ds(sid * _RT_ZERO, _RT_ZERO)])
            pltpu.sync_copy(ones16, ones_v)
        plsc.subcore_barrier()

        def gather(g, b):
            pltpu.async_copy(table.at[src_v.at[g]], bufs[b], gsem[b])

        def wait_gather(g, b):
            pltpu.make_async_copy(table.at[src_v.at[g]], bufs[b], gsem[b]).wait()

        def scatter(g, b):
            pltpu.async_copy(bufs[b], acc.at[dst_v.at[g]], ssem[b], add=True)
            if with_counts:
                pltpu.async_copy(ones_v, cacc.at[dst_v.at[g]], csem[b % 2],
                                 add=True)

        def wait_scatter(g, b):
            pltpu.make_async_copy(bufs[b], acc.at[dst_v.at[g]], ssem[b]).wait()
            if with_counts:
                pltpu.make_async_copy(ones_v, cacc.at[dst_v.at[g]],
                                      csem[b % 2]).wait()

        def step(g, b, first, last):
            wait_gather(g, b)
            scatter(g, b)
            b2 = (b + _D // 2) % _D
            if not first:
                wait_scatter(g - _D // 2, b2)
            if not last:
                gather(g + _D // 2, b2)

        h = _D // 2
        for g in range(h):
            gather(g, g)
        for g in range(h):
            step(g, g, True, False)

        def quad(i, carry):
            base = h + _D * i
            for j in range(_D):
                step(base + j, (h + j) % _D, False, False)
            return carry

        lax.fori_loop(0, (n_chunks - _D) // _D, quad, 0)
        for g in range(n_chunks - h, n_chunks):
            step(g, g % _D, False, True)
        for g in range(n_chunks - h, n_chunks):
            wait_scatter(g, g % _D)

        plsc.subcore_barrier()
        pltpu.sync_copy(
            acc.at[pl.ds(sid * _RT_ZERO, _RT_ZERO)],
            out.at[cid, pl.ds(sid * _RT_ZERO, _RT_ZERO)],
        )
        if with_counts:
            pltpu.sync_copy(
                cacc.at[pl.ds(sid * _RT_ZERO, _RT_ZERO)],
                cout.at[cid, pl.ds(sid * _RT_ZERO, _RT_ZERO)],
            )

    return edge_pass


_B = 1000


def _tc_call(body, n_out, widths, out_widths):
    in_specs = []
    for kind, wdt in widths:
        if kind == "blk":
            in_specs.append(pl.BlockSpec((_B, wdt), lambda i: (i, 0)))
        elif kind == "p":
            in_specs.append(pl.BlockSpec((_NC, _B, wdt), lambda i: (0, i, 0)))
        else:
            r = wdt
            in_specs.append(pl.BlockSpec(r, lambda i: tuple(0 for _ in r)))
    out_specs = [pl.BlockSpec((_B, wo), lambda i: (i, 0)) for wo in out_widths]
    out_shape = [jax.ShapeDtypeStruct((_N, wo), jnp.float32) for wo in out_widths]
    if n_out == 1:
        out_specs, out_shape = out_specs[0], out_shape[0]
    return pl.pallas_call(
        body,
        grid=(_N // _B,),
        in_specs=in_specs,
        out_specs=out_specs,
        out_shape=out_shape,
    )


def _dot(a, b):
    return jnp.dot(a, b, preferred_element_type=jnp.float32)


def _pre_body(x_ref, wl_ref, o_ref):
    o_ref[...] = _dot(x_ref[...], wl_ref[...])


def _combine0_body(p_ref, c_ref, x_ref, wr_ref, bl_ref, wln_ref,
                   h_ref, t_ref, inv_ref):
    p = p_ref[...]
    s = p[0] + p[1]
    c = c_ref[...]
    cnt = (c[0] + c[1])[:, 0:1]
    inv = 1.0 / jnp.maximum(cnt, 1.0)
    h = jnp.maximum(s * inv + bl_ref[...] + _dot(x_ref[...], wr_ref[...]), 0.0)
    h_ref[...] = h
    t_ref[...] = _dot(h, wln_ref[...])
    inv_ref[...] = jnp.broadcast_to(inv, (_B, 8))


def _combine1_body(p_ref, h_ref, wr_ref, bl_ref, inv_ref, wln_ref, h2_ref, t_ref):
    p = p_ref[...]
    s = p[0] + p[1]
    inv = inv_ref[...][:, 0:1]
    h2 = jnp.maximum(s * inv + bl_ref[...] + _dot(h_ref[...], wr_ref[...]), 0.0)
    h2_ref[...] = h2
    t_ref[...] = _dot(h2, wln_ref[...])


def _final_body(p_ref, h_ref, wr_ref, bl_ref, inv_ref, o_ref):
    p = p_ref[...]
    s = (p[0] + p[1])[:, :_C]
    inv = inv_ref[...][:, 0:1]
    logits = s * inv + bl_ref[...] + _dot(h_ref[...], wr_ref[...])
    m = jnp.max(logits, axis=1, keepdims=True)
    z = logits - m
    lse = jnp.log(jnp.sum(jnp.exp(z), axis=1, keepdims=True))
    o_ref[...] = z - lse


def kernel(x, edge_index, Wl0, bl0, Wr0, Wl1, bl1, Wr1, Wl2, bl2, Wr2):
    k0 = 50
    src0 = edge_index[0].reshape(_NW, _E // _NW // k0, k0)
    dst0 = edge_index[1].reshape(_NW, _E // _NW // k0, k0)
    src = edge_index[0].reshape(_NW, _E // _NW // _K, _K)
    dst = edge_index[1].reshape(_NW, _E // _NW // _K, _K)

    wl0t = Wl0.T
    wr0t = Wr0.T
    wl1t = Wl1.T
    wr1t = Wr1.T
    wl2pt = jnp.zeros((_H, 48), jnp.float32).at[:, :_C].set(Wl2.T)
    wr2t = Wr2.T
    bl0r = bl0.reshape(1, _H)
    bl1r = bl1.reshape(1, _H)
    bl2r = bl2.reshape(1, _C)

    z128 = jnp.zeros((_RT_ZERO, 128), jnp.float32)
    z48 = jnp.zeros((_RT_ZERO, 48), jnp.float32)
    z16 = jnp.zeros((_RT_ZERO, 16), jnp.float32)
    ones16 = jnp.zeros((k0, 16), jnp.float32).at[:, 0].set(1.0)

    t0 = _tc_call(
        _pre_body, 1,
        [("blk", 128), ("full", (128, 128))],
        [128],
    )(x, wl0t)
    p0, c0 = _make_edge_pass(128, k0, True)(t0, src0, dst0, z128, z16, ones16)
    h1, t1, inv8 = _tc_call(
        _combine0_body, 3,
        [("p", 128), ("p", 16), ("blk", 128), ("full", (128, 128)),
         ("full", (1, 128)), ("full", (128, 128))],
        [128, 128, 8],
    )(p0, c0, x, wr0t, bl0r, wl1t)

    p1, = _make_edge_pass(128, _K, False)(t1, src, dst, z128)
    h2, t2 = _tc_call(
        _combine1_body, 2,
        [("p", 128), ("blk", 128), ("full", (128, 128)), ("full", (1, 128)),
         ("blk", 8), ("full", (128, 48))],
        [128, 48],
    )(p1, h1, wr1t, bl1r, inv8, wl2pt)

    p2, = _make_edge_pass(48, _K, False)(t2, src, dst, z48)
    out = _tc_call(
        _final_body, 1,
        [("p", 48), ("blk", 128), ("full", (128, 40)), ("full", (1, 40)),
         ("blk", 8)],
        [40],
    )(p2, h2, wr2t, bl2r, inv8)
    return out


# speedup vs baseline: 1.2509x; 1.2509x over previous
import functools

import jax
import jax.numpy as jnp
from jax import lax
from jax.experimental import pallas as pl
from jax.experimental.pallas import tpu as pltpu
from jax.experimental.pallas import tpu_sc as plsc

_N = 10000
_E = 320000
_D = 128
_H = 128
_C = 40

_NC = 2
_NS = 16
_NW = _NC * _NS
_K = 100

_RT_ZERO = 640
_NPAD = _NS * _RT_ZERO


def _make_edge_pass(w, k, with_counts):
    e_per_tile = _E // _NW
    n_chunks = e_per_tile // k
    assert n_chunks % 2 == 0 and e_per_tile % k == 0
    mesh = plsc.VectorSubcoreMesh(core_axis_name="c", subcore_axis_name="s")

    out_type = [jax.ShapeDtypeStruct((_NC, _NPAD, w), jnp.float32)]
    scratch = [
        pltpu.VMEM_SHARED((_NPAD, w), jnp.float32),
        pltpu.VMEM((n_chunks, k), jnp.int32),
        pltpu.VMEM((n_chunks, k), jnp.int32),
        pltpu.VMEM((k, w), jnp.float32),
        pltpu.VMEM((k, w), jnp.float32),
        pltpu.SemaphoreType.DMA,
        pltpu.SemaphoreType.DMA,
    ]
    if with_counts:
        out_type.append(jax.ShapeDtypeStruct((_NC, _NPAD, 16), jnp.float32))
        scratch += [
            pltpu.VMEM_SHARED((_NPAD, 16), jnp.float32),
            pltpu.VMEM((k, 16), jnp.float32),
            pltpu.SemaphoreType.DMA,
            pltpu.SemaphoreType.DMA,
        ]

    @functools.partial(
        pl.kernel,
        out_type=out_type,
        mesh=mesh,
        scratch_types=scratch,
        compiler_params=pltpu.CompilerParams(use_tc_tiling_on_sc=False),
    )
    def edge_pass(table, src2, dst2, zrows, *rest):
        if with_counts:
            (zrows16, ones16, out, cout, acc, src_v, dst_v,
             buf_a, buf_b, sem_a, sem_b, cacc, ones_v, csem0, csem1) = rest
            csem = [csem0, csem1]
        else:
            out, acc, src_v, dst_v, buf_a, buf_b, sem_a, sem_b = rest
        cid = lax.axis_index("c")
        sid = lax.axis_index("s")
        wid = cid * _NS + sid
        pltpu.sync_copy(zrows, acc.at[pl.ds(sid * _RT_ZERO, _RT_ZERO)])
        pltpu.sync_copy(src2.at[wid], src_v)
        pltpu.sync_copy(dst2.at[wid], dst_v)
        if with_counts:
            pltpu.sync_copy(zrows16, cacc.at[pl.ds(sid * _RT_ZERO, _RT_ZERO)])
            pltpu.sync_copy(ones16, ones_v)
        plsc.subcore_barrier()

        pltpu.async_copy(table.at[src_v.at[0]], buf_a, sem_a)
        pltpu.async_copy(table.at[src_v.at[1]], buf_b, sem_b)

        def half_step(g, buf, sem, cs):
            pltpu.make_async_copy(table.at[src_v.at[g]], buf, sem).wait()
            pltpu.sync_copy(buf, acc.at[dst_v.at[g]], add=True)
            if with_counts:
                pltpu.async_copy(ones_v, cacc.at[dst_v.at[g]], cs, add=True)

                @pl.when(g >= 2)
                def _():
                    pltpu.make_async_copy(
                        ones_v, cacc.at[dst_v.at[g - 2]], cs).wait()

            @pl.when(g + 2 < n_chunks)
            def _():
                pltpu.async_copy(table.at[src_v.at[g + 2]], buf, sem)

        def pair(i, carry):
            half_step(2 * i, buf_a, sem_a, csem0 if with_counts else None)
            half_step(2 * i + 1, buf_b, sem_b, csem1 if with_counts else None)
            return carry

        lax.fori_loop(0, n_chunks // 2, pair, 0)
        if with_counts:
            for g in (n_chunks - 2, n_chunks - 1):
                pltpu.make_async_copy(
                    ones_v, cacc.at[dst_v.at[g]], csem[g % 2]).wait()

        plsc.subcore_barrier()
        pltpu.sync_copy(
            acc.at[pl.ds(sid * _RT_ZERO, _RT_ZERO)],
            out.at[cid, pl.ds(sid * _RT_ZERO, _RT_ZERO)],
        )
        if with_counts:
            pltpu.sync_copy(
                cacc.at[pl.ds(sid * _RT_ZERO, _RT_ZERO)],
                cout.at[cid, pl.ds(sid * _RT_ZERO, _RT_ZERO)],
            )

    return edge_pass


_B = 1000


def _tc_call(body, n_out, widths, out_widths):
    in_specs = []
    for kind, wdt in widths:
        if kind == "blk":
            in_specs.append(pl.BlockSpec((_B, wdt), lambda i: (i, 0)))
        elif kind == "p":
            in_specs.append(pl.BlockSpec((_NC, _B, wdt), lambda i: (0, i, 0)))
        else:
            r = wdt
            in_specs.append(pl.BlockSpec(r, lambda i: tuple(0 for _ in r)))
    out_specs = [pl.BlockSpec((_B, wo), lambda i: (i, 0)) for wo in out_widths]
    out_shape = [jax.ShapeDtypeStruct((_N, wo), jnp.float32) for wo in out_widths]
    if n_out == 1:
        out_specs, out_shape = out_specs[0], out_shape[0]
    return pl.pallas_call(
        body,
        grid=(_N // _B,),
        in_specs=in_specs,
        out_specs=out_specs,
        out_shape=out_shape,
    )


def _dot(a, b):
    return jnp.dot(a, b, preferred_element_type=jnp.float32)


def _pre_body(x_ref, wl_ref, o_ref):
    o_ref[...] = _dot(x_ref[...], wl_ref[...])


def _combine0_body(p_ref, c_ref, x_ref, wr_ref, bl_ref, wln_ref,
                   h_ref, t_ref, inv_ref):
    p = p_ref[...]
    s = p[0] + p[1]
    c = c_ref[...]
    cnt = (c[0] + c[1])[:, 0:1]
    inv = 1.0 / jnp.maximum(cnt, 1.0)
    h = jnp.maximum(s * inv + bl_ref[...] + _dot(x_ref[...], wr_ref[...]), 0.0)
    h_ref[...] = h
    t_ref[...] = _dot(h, wln_ref[...])
    inv_ref[...] = jnp.broadcast_to(inv, (_B, 8))


def _combine1_body(p_ref, h_ref, wr_ref, bl_ref, inv_ref, wln_ref, h2_ref, t_ref):
    p = p_ref[...]
    s = p[0] + p[1]
    inv = inv_ref[...][:, 0:1]
    h2 = jnp.maximum(s * inv + bl_ref[...] + _dot(h_ref[...], wr_ref[...]), 0.0)
    h2_ref[...] = h2
    t_ref[...] = _dot(h2, wln_ref[...])


def _final_body(p_ref, h_ref, wr_ref, bl_ref, inv_ref, o_ref):
    p = p_ref[...]
    s = (p[0] + p[1])[:, :_C]
    inv = inv_ref[...][:, 0:1]
    logits = s * inv + bl_ref[...] + _dot(h_ref[...], wr_ref[...])
    m = jnp.max(logits, axis=1, keepdims=True)
    z = logits - m
    lse = jnp.log(jnp.sum(jnp.exp(z), axis=1, keepdims=True))
    o_ref[...] = z - lse


def kernel(x, edge_index, Wl0, bl0, Wr0, Wl1, bl1, Wr1, Wl2, bl2, Wr2):
    k0 = 50
    src0 = edge_index[0].reshape(_NW, _E // _NW // k0, k0)
    dst0 = edge_index[1].reshape(_NW, _E // _NW // k0, k0)
    src = edge_index[0].reshape(_NW, _E // _NW // _K, _K)
    dst = edge_index[1].reshape(_NW, _E // _NW // _K, _K)

    wl0t = Wl0.T
    wr0t = Wr0.T
    wl1t = Wl1.T
    wr1t = Wr1.T
    wl2pt = jnp.zeros((_H, 48), jnp.float32).at[:, :_C].set(Wl2.T)
    wr2t = Wr2.T
    bl0r = bl0.reshape(1, _H)
    bl1r = bl1.reshape(1, _H)
    bl2r = bl2.reshape(1, _C)

    z128 = jnp.zeros((_RT_ZERO, 128), jnp.float32)
    z48 = jnp.zeros((_RT_ZERO, 48), jnp.float32)
    z16 = jnp.zeros((_RT_ZERO, 16), jnp.float32)
    ones16 = jnp.zeros((k0, 16), jnp.float32).at[:, 0].set(1.0)

    t0 = _tc_call(
        _pre_body, 1,
        [("blk", 128), ("full", (128, 128))],
        [128],
    )(x, wl0t)
    p0, c0 = _make_edge_pass(128, k0, True)(t0, src0, dst0, z128, z16, ones16)
    h1, t1, inv8 = _tc_call(
        _combine0_body, 3,
        [("p", 128), ("p", 16), ("blk", 128), ("full", (128, 128)),
         ("full", (1, 128)), ("full", (128, 128))],
        [128, 128, 8],
    )(p0, c0, x, wr0t, bl0r, wl1t)

    p1, = _make_edge_pass(128, _K, False)(t1, src, dst, z128)
    h2, t2 = _tc_call(
        _combine1_body, 2,
        [("p", 128), ("blk", 128), ("full", (128, 128)), ("full", (1, 128)),
         ("blk", 8), ("full", (128, 48))],
        [128, 48],
    )(p1, h1, wr1t, bl1r, inv8, wl2pt)

    p2, = _make_edge_pass(48, _K, False)(t2, src, dst, z48)
    out = _tc_call(
        _final_body, 1,
        [("p", 48), ("blk", 128), ("full", (128, 40)), ("full", (1, 40)),
         ("blk", 8)],
        [40],
    )(p2, h2, wr2t, bl2r, inv8)
    return out


# speedup vs baseline: 1.4103x; 1.1274x over previous
import functools

import jax
import jax.numpy as jnp
from jax import lax
from jax.experimental import pallas as pl
from jax.experimental.pallas import tpu as pltpu
from jax.experimental.pallas import tpu_sc as plsc

_N = 10000
_E = 320000
_D = 128
_H = 128
_C = 40

_NC = 2
_NS = 16
_NW = _NC * _NS
_K = 100

_RT_ZERO = 640
_NPAD = _NS * _RT_ZERO


def _make_edge_pass(w, k, with_counts):
    e_per_tile = _E // _NW
    n_chunks = e_per_tile // k
    assert n_chunks % 4 == 0 and e_per_tile % k == 0 and k % 4 == 0
    mesh = plsc.VectorSubcoreMesh(core_axis_name="c", subcore_axis_name="s")

    out_type = [jax.ShapeDtypeStruct((_NC, _NPAD, w), jnp.float32)]
    scratch = [
        pltpu.VMEM_SHARED((_NPAD, w), jnp.float32),
        pltpu.VMEM((n_chunks, k), jnp.int32),
        pltpu.VMEM((k, w), jnp.float32),
        pltpu.VMEM((k, w), jnp.float32),
        pltpu.SemaphoreType.DMA,
        pltpu.SemaphoreType.DMA,
    ]
    scratch += [pltpu.VMEM((k,), jnp.int32) for _ in range(8)]
    if with_counts:
        out_type.append(jax.ShapeDtypeStruct((_NC, _NPAD, 16), jnp.float32))
        scratch += [
            pltpu.VMEM_SHARED((_NPAD, 16), jnp.float32),
            pltpu.VMEM((k, 16), jnp.float32),
            pltpu.SemaphoreType.DMA,
            pltpu.SemaphoreType.DMA,
        ]

    offs = list(range(0, k - 15, 16))
    if offs[-1] + 16 < k:
        offs.append(k - 16)

    @functools.partial(
        pl.kernel,
        out_type=out_type,
        mesh=mesh,
        scratch_types=scratch,
        compiler_params=pltpu.CompilerParams(use_tc_tiling_on_sc=False),
    )
    def edge_pass(table, packed, zrows, *rest):
        if with_counts:
            (zrows16, ones16, out, cout, acc, pk_v, buf_a, buf_b,
             sem_a, sem_b, *tail) = rest
            src_u = tail[:4]
            dst_u = tail[4:8]
            cacc, ones_v, csem0, csem1 = tail[8:]
            csem = [csem0, csem1]
        else:
            out, acc, pk_v, buf_a, buf_b, sem_a, sem_b, *tail = rest
            src_u = tail[:4]
            dst_u = tail[4:8]
        bufs = [buf_a, buf_b]
        sems = [sem_a, sem_b]
        cid = lax.axis_index("c")
        sid = lax.axis_index("s")
        wid = cid * _NS + sid
        pltpu.sync_copy(zrows, acc.at[pl.ds(sid * _RT_ZERO, _RT_ZERO)])
        pltpu.sync_copy(packed.at[wid], pk_v)
        if with_counts:
            pltpu.sync_copy(zrows16, cacc.at[pl.ds(sid * _RT_ZERO, _RT_ZERO)])
            pltpu.sync_copy(ones16, ones_v)
        plsc.subcore_barrier()

        def unpack(g, slot):
            for off in offs:
                v = pk_v[g, pl.ds(off, 16)]
                src_u[slot][pl.ds(off, 16)] = v & 16383
                dst_u[slot][pl.ds(off, 16)] = lax.shift_right_logical(v, 14)

        def gather(g, slot, p):
            pltpu.async_copy(table.at[src_u[slot]], bufs[p], sems[p])

        def step(g, slot, first=False, last=False):
            p = slot % 2
            pltpu.make_async_copy(table.at[src_u[slot]], bufs[p], sems[p]).wait()
            pltpu.sync_copy(bufs[p], acc.at[dst_u[slot]], add=True)
            if with_counts:
                pltpu.async_copy(ones_v, cacc.at[dst_u[slot]], csem[p],
                                 add=True)
                if not first:
                    s2 = (slot + 2) % 4
                    pltpu.make_async_copy(ones_v, cacc.at[dst_u[s2]],
                                          csem[p]).wait()
            if not last:
                s2 = (slot + 2) % 4
                unpack(g + 2, s2)
                gather(g + 2, s2, p)

        unpack(0, 0)
        unpack(1, 1)
        gather(0, 0, 0)
        gather(1, 1, 1)
        step(0, 0, first=True)
        step(1, 1, first=True)

        def quad(i, carry):
            base = 2 + 4 * i
            for j in range(4):
                step(base + j, (2 + j) % 4)
            return carry

        lax.fori_loop(0, (n_chunks - 4) // 4, quad, 0)
        step(n_chunks - 2, (n_chunks - 2) % 4, last=True)
        step(n_chunks - 1, (n_chunks - 1) % 4, last=True)
        if with_counts:
            for g in (n_chunks - 2, n_chunks - 1):
                pltpu.make_async_copy(
                    ones_v, cacc.at[dst_u[g % 4]], csem[g % 2]).wait()

        plsc.subcore_barrier()
        pltpu.sync_copy(
            acc.at[pl.ds(sid * _RT_ZERO, _RT_ZERO)],
            out.at[cid, pl.ds(sid * _RT_ZERO, _RT_ZERO)],
        )
        if with_counts:
            pltpu.sync_copy(
                cacc.at[pl.ds(sid * _RT_ZERO, _RT_ZERO)],
                cout.at[cid, pl.ds(sid * _RT_ZERO, _RT_ZERO)],
            )

    return edge_pass


_B = 1000


def _tc_call(body, n_out, widths, out_widths):
    in_specs = []
    for kind, wdt in widths:
        if kind == "blk":
            in_specs.append(pl.BlockSpec((_B, wdt), lambda i: (i, 0)))
        elif kind == "p":
            in_specs.append(pl.BlockSpec((_NC, _B, wdt), lambda i: (0, i, 0)))
        else:
            r = wdt
            in_specs.append(pl.BlockSpec(r, lambda i: tuple(0 for _ in r)))
    out_specs = [pl.BlockSpec((_B, wo), lambda i: (i, 0)) for wo in out_widths]
    out_shape = [jax.ShapeDtypeStruct((_N, wo), jnp.float32) for wo in out_widths]
    if n_out == 1:
        out_specs, out_shape = out_specs[0], out_shape[0]
    return pl.pallas_call(
        body,
        grid=(_N // _B,),
        in_specs=in_specs,
        out_specs=out_specs,
        out_shape=out_shape,
    )


def _dot(a, b):
    return jnp.dot(a, b, preferred_element_type=jnp.float32)


def _pre_body(x_ref, wl_ref, o_ref):
    o_ref[...] = _dot(x_ref[...], wl_ref[...])


def _combine0_body(p_ref, c_ref, x_ref, wr_ref, bl_ref, wln_ref,
                   h_ref, t_ref, inv_ref):
    p = p_ref[...]
    s = p[0] + p[1]
    c = c_ref[...]
    cnt = (c[0] + c[1])[:, 0:1]
    inv = 1.0 / jnp.maximum(cnt, 1.0)
    h = jnp.maximum(s * inv + bl_ref[...] + _dot(x_ref[...], wr_ref[...]), 0.0)
    h_ref[...] = h
    t_ref[...] = _dot(h, wln_ref[...])
    inv_ref[...] = jnp.broadcast_to(inv, (_B, 8))


def _combine1_body(p_ref, h_ref, wr_ref, bl_ref, inv_ref, wln_ref, h2_ref, t_ref):
    p = p_ref[...]
    s = p[0] + p[1]
    inv = inv_ref[...][:, 0:1]
    h2 = jnp.maximum(s * inv + bl_ref[...] + _dot(h_ref[...], wr_ref[...]), 0.0)
    h2_ref[...] = h2
    t_ref[...] = _dot(h2, wln_ref[...])


def _final_body(p_ref, h_ref, wr_ref, bl_ref, inv_ref, o_ref):
    p = p_ref[...]
    s = (p[0] + p[1])[:, :_C]
    inv = inv_ref[...][:, 0:1]
    logits = s * inv + bl_ref[...] + _dot(h_ref[...], wr_ref[...])
    m = jnp.max(logits, axis=1, keepdims=True)
    z = logits - m
    lse = jnp.log(jnp.sum(jnp.exp(z), axis=1, keepdims=True))
    o_ref[...] = z - lse


def kernel(x, edge_index, Wl0, bl0, Wr0, Wl1, bl1, Wr1, Wl2, bl2, Wr2):
    packed = (edge_index[0] | (edge_index[1] << 14)).reshape(
        _NW, _E // _NW // _K, _K)

    wl0t = Wl0.T
    wr0t = Wr0.T
    wl1t = Wl1.T
    wr1t = Wr1.T
    wl2pt = jnp.zeros((_H, 48), jnp.float32).at[:, :_C].set(Wl2.T)
    wr2t = Wr2.T
    bl0r = bl0.reshape(1, _H)
    bl1r = bl1.reshape(1, _H)
    bl2r = bl2.reshape(1, _C)

    z128 = jnp.zeros((_RT_ZERO, 128), jnp.float32)
    z48 = jnp.zeros((_RT_ZERO, 48), jnp.float32)
    z16 = jnp.zeros((_RT_ZERO, 16), jnp.float32)
    ones16 = jnp.zeros((_K, 16), jnp.float32).at[:, 0].set(1.0)

    t0 = _tc_call(
        _pre_body, 1,
        [("blk", 128), ("full", (128, 128))],
        [128],
    )(x, wl0t)
    p0, c0 = _make_edge_pass(128, _K, True)(t0, packed, z128, z16, ones16)
    h1, t1, inv8 = _tc_call(
        _combine0_body, 3,
        [("p", 128), ("p", 16), ("blk", 128), ("full", (128, 128)),
         ("full", (1, 128)), ("full", (128, 128))],
        [128, 128, 8],
    )(p0, c0, x, wr0t, bl0r, wl1t)

    p1, = _make_edge_pass(128, _K, False)(t1, packed, z128)
    h2, t2 = _tc_call(
        _combine1_body, 2,
        [("p", 128), ("blk", 128), ("full", (128, 128)), ("full", (1, 128)),
         ("blk", 8), ("full", (128, 48))],
        [128, 48],
    )(p1, h1, wr1t, bl1r, inv8, wl2pt)

    p2, = _make_edge_pass(48, _K, False)(t2, packed, z48)
    out = _tc_call(
        _final_body, 1,
        [("p", 48), ("blk", 128), ("full", (128, 40)), ("full", (1, 40)),
         ("blk", 8)],
        [40],
    )(p2, h2, wr2t, bl2r, inv8)
    return out


# speedup vs baseline: 1.4511x; 1.0289x over previous
import functools

import jax
import jax.numpy as jnp
from jax import lax
from jax.experimental import pallas as pl
from jax.experimental.pallas import tpu as pltpu
from jax.experimental.pallas import tpu_sc as plsc

_N = 10000
_E = 320000
_D = 128
_H = 128
_C = 40

_NC = 2
_NS = 16
_NW = _NC * _NS
_K = 100

_RT_ZERO = 640
_NPAD = _NS * _RT_ZERO


def _make_edge_pass(w, k, with_counts):
    e_per_tile = _E // _NW
    n_chunks = e_per_tile // k
    assert n_chunks % 4 == 0 and e_per_tile % k == 0 and k % 4 == 0
    mesh = plsc.VectorSubcoreMesh(core_axis_name="c", subcore_axis_name="s")

    out_type = [jax.ShapeDtypeStruct((_NC, _NPAD, w), jnp.float32)]
    scratch = [
        pltpu.VMEM_SHARED((_NPAD, w), jnp.float32),
        pltpu.VMEM((e_per_tile,), jnp.int32),
        pltpu.VMEM((k, w), jnp.float32),
        pltpu.VMEM((k, w), jnp.float32),
        pltpu.SemaphoreType.DMA,
        pltpu.SemaphoreType.DMA,
    ]
    scratch += [pltpu.VMEM((k,), jnp.int32) for _ in range(8)]
    if with_counts:
        out_type.append(jax.ShapeDtypeStruct((_NC, _NPAD, 16), jnp.float32))
        scratch += [
            pltpu.VMEM_SHARED((_NPAD, 16), jnp.float32),
            pltpu.VMEM((k, 16), jnp.float32),
            pltpu.SemaphoreType.DMA,
            pltpu.SemaphoreType.DMA,
        ]

    offs = list(range(0, k - 15, 16))
    if offs[-1] + 16 < k:
        offs.append(k - 16)

    @functools.partial(
        pl.kernel,
        out_type=out_type,
        mesh=mesh,
        scratch_types=scratch,
        compiler_params=pltpu.CompilerParams(use_tc_tiling_on_sc=False),
    )
    def edge_pass(table, packed, zrows, *rest):
        if with_counts:
            (zrows16, ones16, out, cout, acc, pk_v, buf_a, buf_b,
             sem_a, sem_b, *tail) = rest
            src_u = tail[:4]
            dst_u = tail[4:8]
            cacc, ones_v, csem0, csem1 = tail[8:]
            csem = [csem0, csem1]
        else:
            out, acc, pk_v, buf_a, buf_b, sem_a, sem_b, *tail = rest
            src_u = tail[:4]
            dst_u = tail[4:8]
        bufs = [buf_a, buf_b]
        sems = [sem_a, sem_b]
        cid = lax.axis_index("c")
        sid = lax.axis_index("s")
        wid = cid * _NS + sid
        pltpu.sync_copy(zrows, acc.at[pl.ds(sid * _RT_ZERO, _RT_ZERO)])
        pltpu.sync_copy(packed.at[pl.ds(wid * e_per_tile, e_per_tile)], pk_v)
        if with_counts:
            pltpu.sync_copy(zrows16, cacc.at[pl.ds(sid * _RT_ZERO, _RT_ZERO)])
            pltpu.sync_copy(ones16, ones_v)
        plsc.subcore_barrier()

        def unpack(g, slot):
            for off in offs:
                v = pk_v[pl.ds(g * k + off, 16)]
                src_u[slot][pl.ds(off, 16)] = v & 16383
                dst_u[slot][pl.ds(off, 16)] = lax.shift_right_logical(v, 14)

        def gather(g, slot, p):
            pltpu.async_copy(table.at[src_u[slot]], bufs[p], sems[p])

        def step(g, slot, first=False, last=False):
            p = slot % 2
            pltpu.make_async_copy(table.at[src_u[slot]], bufs[p], sems[p]).wait()
            pltpu.sync_copy(bufs[p], acc.at[dst_u[slot]], add=True)
            if with_counts:
                pltpu.async_copy(ones_v, cacc.at[dst_u[slot]], csem[p],
                                 add=True)
                if not first:
                    s2 = (slot + 2) % 4
                    pltpu.make_async_copy(ones_v, cacc.at[dst_u[s2]],
                                          csem[p]).wait()
            if not last:
                s2 = (slot + 2) % 4
                unpack(g + 2, s2)
                gather(g + 2, s2, p)

        unpack(0, 0)
        unpack(1, 1)
        gather(0, 0, 0)
        gather(1, 1, 1)
        step(0, 0, first=True)
        step(1, 1, first=True)

        def quad(i, carry):
            base = 2 + 4 * i
            for j in range(4):
                step(base + j, (2 + j) % 4)
            return carry

        lax.fori_loop(0, (n_chunks - 4) // 4, quad, 0)
        step(n_chunks - 2, (n_chunks - 2) % 4, last=True)
        step(n_chunks - 1, (n_chunks - 1) % 4, last=True)
        if with_counts:
            for g in (n_chunks - 2, n_chunks - 1):
                pltpu.make_async_copy(
                    ones_v, cacc.at[dst_u[g % 4]], csem[g % 2]).wait()

        plsc.subcore_barrier()
        pltpu.sync_copy(
            acc.at[pl.ds(sid * _RT_ZERO, _RT_ZERO)],
            out.at[cid, pl.ds(sid * _RT_ZERO, _RT_ZERO)],
        )
        if with_counts:
            pltpu.sync_copy(
                cacc.at[pl.ds(sid * _RT_ZERO, _RT_ZERO)],
                cout.at[cid, pl.ds(sid * _RT_ZERO, _RT_ZERO)],
            )

    return edge_pass


_B = 2000


def _tc_call(body, n_out, widths, out_widths):
    in_specs = []
    for kind, wdt in widths:
        if kind == "blk":
            in_specs.append(pl.BlockSpec((_B, wdt), lambda i: (i, 0)))
        elif kind == "p":
            in_specs.append(pl.BlockSpec((_NC, _B, wdt), lambda i: (0, i, 0)))
        else:
            r = wdt
            in_specs.append(pl.BlockSpec(r, lambda i: tuple(0 for _ in r)))
    out_specs = [pl.BlockSpec((_B, wo), lambda i: (i, 0)) for wo in out_widths]
    out_shape = [jax.ShapeDtypeStruct((_N, wo), jnp.float32) for wo in out_widths]
    if n_out == 1:
        out_specs, out_shape = out_specs[0], out_shape[0]
    return pl.pallas_call(
        body,
        grid=(_N // _B,),
        in_specs=in_specs,
        out_specs=out_specs,
        out_shape=out_shape,
    )


def _dot(a, b):
    return lax.dot_general(a, b, (((1,), (1,)), ((), ())),
                           preferred_element_type=jnp.float32)


def _pre_body(x_ref, wl_ref, o_ref):
    o_ref[...] = _dot(x_ref[...], wl_ref[...])


def _combine0_body(p_ref, c_ref, x_ref, wr_ref, bl_ref, wln_ref,
                   h_ref, t_ref, inv_ref):
    p = p_ref[...]
    s = p[0] + p[1]
    c = c_ref[...]
    cnt = (c[0] + c[1])[:, 0:1]
    inv = 1.0 / jnp.maximum(cnt, 1.0)
    h = jnp.maximum(s * inv + bl_ref[...] + _dot(x_ref[...], wr_ref[...]), 0.0)
    h_ref[...] = h
    t_ref[...] = _dot(h, wln_ref[...])
    inv_ref[...] = jnp.broadcast_to(inv, (_B, 8))


def _combine1_body(p_ref, h_ref, wr_ref, bl_ref, inv_ref, wln_ref, h2_ref, t_ref):
    p = p_ref[...]
    s = p[0] + p[1]
    inv = inv_ref[...][:, 0:1]
    h2 = jnp.maximum(s * inv + bl_ref[...] + _dot(h_ref[...], wr_ref[...]), 0.0)
    h2_ref[...] = h2
    t_ref[...] = _dot(h2, wln_ref[...])


def _final_body(p_ref, h_ref, wr_ref, bl_ref, inv_ref, o_ref):
    p = p_ref[...]
    s = (p[0] + p[1])[:, :_C]
    inv = inv_ref[...][:, 0:1]
    logits = s * inv + bl_ref[...] + _dot(h_ref[...], wr_ref[...])
    m = jnp.max(logits, axis=1, keepdims=True)
    z = logits - m
    lse = jnp.log(jnp.sum(jnp.exp(z), axis=1, keepdims=True))
    o_ref[...] = z - lse


def kernel(x, edge_index, Wl0, bl0, Wr0, Wl1, bl1, Wr1, Wl2, bl2, Wr2):
    packed = edge_index[0] | (edge_index[1] << 14)

    wl2p = jnp.zeros((48, _H), jnp.float32).at[:_C].set(Wl2)
    bl0r = bl0.reshape(1, _H)
    bl1r = bl1.reshape(1, _H)
    bl2r = bl2.reshape(1, _C)

    z128 = jnp.zeros((_RT_ZERO, 128), jnp.float32)
    z48 = jnp.zeros((_RT_ZERO, 48), jnp.float32)
    z16 = jnp.zeros((_RT_ZERO, 16), jnp.float32)
    ones16 = jnp.zeros((_K, 16), jnp.float32).at[:, 0].set(1.0)

    t0 = _tc_call(
        _pre_body, 1,
        [("blk", 128), ("full", (128, 128))],
        [128],
    )(x, Wl0)
    p0, c0 = _make_edge_pass(128, _K, True)(t0, packed, z128, z16, ones16)
    h1, t1, inv8 = _tc_call(
        _combine0_body, 3,
        [("p", 128), ("p", 16), ("blk", 128), ("full", (128, 128)),
         ("full", (1, 128)), ("full", (128, 128))],
        [128, 128, 8],
    )(p0, c0, x, Wr0, bl0r, Wl1)

    p1, = _make_edge_pass(128, _K, False)(t1, packed, z128)
    h2, t2 = _tc_call(
        _combine1_body, 2,
        [("p", 128), ("blk", 128), ("full", (128, 128)), ("full", (1, 128)),
         ("blk", 8), ("full", (48, 128))],
        [128, 48],
    )(p1, h1, Wr1, bl1r, inv8, wl2p)

    p2, = _make_edge_pass(48, _K, False)(t2, packed, z48)
    out = _tc_call(
        _final_body, 1,
        [("p", 48), ("blk", 128), ("full", (40, 128)), ("full", (1, 40)),
         ("blk", 8)],
        [40],
    )(p2, h2, Wr2, bl2r, inv8)
    return out
